# baseline (speedup 1.0000x reference)
import functools

import jax
import jax.numpy as jnp
from jax import lax
from jax.experimental import pallas as pl
from jax.experimental.pallas import tpu as pltpu
from jax.experimental.pallas import tpu_sc as plsc

N = 10000
E = 160000
D = 256
H = 16
DH = 128

NC = 2
NS = 16
NW = NC * NS
CHUNK = 125
NROWS = E // CHUNK
ROWS_PER_W = NROWS // NW
K3_RPS = NROWS // NS
NPAD = 10240
NPS = NPAD // NS
OUTCH = 128
ROW_BLK = 1000


def _k1_body(h_ref, winT_ref, bin_ref, wuv_ref, buv_ref,
             h1a_ref, h1b_ref, su_ref, sv_ref, m_ref, acc_ref):
    i = pl.program_id(0)
    h1 = jnp.dot(h_ref[...], winT_ref[...],
                 preferred_element_type=jnp.float32) + bin_ref[...]
    h1a_ref[...] = h1[:, :DH]
    h1b_ref[...] = h1[:, DH:]
    suv = jnp.dot(h1, wuv_ref[...],
                  preferred_element_type=jnp.float32) + buv_ref[...]
    su_ref[...] = suv[:, :H]
    sv_ref[...] = suv[:, H:]
    blkmax = jnp.max(suv, axis=0, keepdims=True)

    @pl.when(i == 0)
    def _():
        acc_ref[...] = blkmax

    @pl.when(i > 0)
    def _():
        acc_ref[...] = jnp.maximum(acc_ref[...], blkmax)

    m = acc_ref[...]
    mm = m[:, :H] + m[:, H:]
    m_ref[...] = jnp.where(mm >= 0, mm, 0.2 * mm)


def _k1(h, winT, bin2, wuv, buv):
    nblk = N // ROW_BLK
    return pl.pallas_call(
        _k1_body,
        grid=(nblk,),
        in_specs=[
            pl.BlockSpec((ROW_BLK, D), lambda i: (i, 0)),
            pl.BlockSpec((D, D), lambda i: (0, 0)),
            pl.BlockSpec((1, D), lambda i: (0, 0)),
            pl.BlockSpec((D, 2 * H), lambda i: (0, 0)),
            pl.BlockSpec((1, 2 * H), lambda i: (0, 0)),
        ],
        out_specs=[
            pl.BlockSpec((ROW_BLK, DH), lambda i: (i, 0)),
            pl.BlockSpec((ROW_BLK, DH), lambda i: (i, 0)),
            pl.BlockSpec((ROW_BLK, H), lambda i: (i, 0)),
            pl.BlockSpec((ROW_BLK, H), lambda i: (i, 0)),
            pl.BlockSpec((1, H), lambda i: (0, 0)),
        ],
        out_shape=[
            jax.ShapeDtypeStruct((N, DH), jnp.float32),
            jax.ShapeDtypeStruct((N, DH), jnp.float32),
            jax.ShapeDtypeStruct((N, H), jnp.float32),
            jax.ShapeDtypeStruct((N, H), jnp.float32),
            jax.ShapeDtypeStruct((1, H), jnp.float32),
        ],
        scratch_shapes=[pltpu.VMEM((1, 2 * H), jnp.float32)],
    )(h, winT, bin2, wuv, buv)


def _k2_body(src2, dst2, su_hbm, sv_hbm, m_hbm, z16_hbm,
             ex_hbm, den_hbm,
             sidx0, sidx1, sidx2, sidx3, didx0, didx1, didx2, didx3,
             surow0, svrow0, p0, surow1, svrow1, p1,
             mvec, den_sh,
             isem0, isem1, isem2, isem3, gsem0, gsem1, wsem0, wsem1):
    c = lax.axis_index("c")
    s = lax.axis_index("s")
    wid = c * NS + s
    base = wid * ROWS_PER_W
    pltpu.sync_copy(z16_hbm, den_sh.at[pl.ds(s * NPS, NPS)])
    pltpu.sync_copy(m_hbm.at[0], mvec)
    plsc.subcore_barrier()
    mval = mvec[...]

    sidx_b = (sidx0, sidx1, sidx2, sidx3)
    didx_b = (didx0, didx1, didx2, didx3)
    isem_b = (isem0, isem1, isem2, isem3)
    surow_b = (surow0, surow1)
    svrow_b = (svrow0, svrow1)
    p_b = (p0, p1)
    gsem_b = (gsem0, gsem1)
    wsem_b = (wsem0, wsem1)

    def idx_load(j, q):
        pltpu.async_copy(src2.at[base + j, 0], sidx_b[q], isem_b[q])
        pltpu.async_copy(dst2.at[base + j, 0], didx_b[q], isem_b[q])

    def wait_idx(q):
        pltpu.make_async_copy(src2.at[base, 0], sidx_b[q], isem_b[q]).wait()
        pltpu.make_async_copy(dst2.at[base, 0], didx_b[q], isem_b[q]).wait()

    def gathers(d, q):
        pltpu.async_copy(su_hbm.at[sidx_b[q]], surow_b[d], gsem_b[d])
        pltpu.async_copy(sv_hbm.at[didx_b[q]], svrow_b[d], gsem_b[d])

    def wait_g(d, q):
        pltpu.make_async_copy(su_hbm.at[sidx_b[q]], surow_b[d],
                              gsem_b[d]).wait()
        pltpu.make_async_copy(sv_hbm.at[didx_b[q]], svrow_b[d],
                              gsem_b[d]).wait()

    def stores(j, d, q):
        pltpu.async_copy(p_b[d], ex_hbm.at[base + j], wsem_b[d])
        pltpu.sync_copy(p_b[d], den_sh.at[didx_b[q]], add=True)

    def wait_w(d):
        pltpu.make_async_copy(p_b[d], ex_hbm.at[base], wsem_b[d]).wait()

    idx_load(0, 0)
    idx_load(1, 1)
    wait_idx(0)
    gathers(0, 0)

    def quad(jj, _):
        for t in range(4):
            j = 4 * jj + t
            d = t % 2
            u = 1 - d
            surow, svrow, p = surow_b[d], svrow_b[d], p_b[d]
            wait_g(d, t)
            if t < 2:
                @pl.when(jj > 0)
                def _():
                    wait_w(d)
            else:
                wait_w(d)

            @plsc.parallel_loop(0, CHUNK, unroll=5)
            def _(e):
                v = surow[e] + svrow[e]
                v = jnp.where(v >= 0, v, 0.2 * v) - mval
                p[e] = jnp.exp(v)

            if t == 0:
                wait_idx(1)
                gathers(u, 1)
                idx_load(j + 2, 2)
            elif t == 3:
                @pl.when(jj < ROWS_PER_W // 4 - 1)
                def _():
                    wait_idx(0)
                    gathers(u, 0)
                    idx_load(j + 2, 1)
            else:
                wait_idx(t + 1)
                gathers(u, t + 1)
                if t == 2:
                    @pl.when(jj < ROWS_PER_W // 4 - 1)
                    def _():
                        idx_load(j + 2, 0)
                else:
                    idx_load(j + 2, 3)

            stores(j, d, t)
        return 0

    lax.fori_loop(0, ROWS_PER_W // 4, quad, 0)
    wait_w(0)
    wait_w(1)
    plsc.subcore_barrier()
    pltpu.sync_copy(den_sh.at[pl.ds(s * NPS, NPS)],
                    den_hbm.at[c, pl.ds(s * NPS, NPS)])


@functools.lru_cache(maxsize=None)
def _k2():
    mesh = plsc.VectorSubcoreMesh(core_axis_name="c", subcore_axis_name="s")
    return functools.partial(
        pl.kernel,
        out_type=[
            jax.ShapeDtypeStruct((NROWS, CHUNK, H), jnp.float32),
            jax.ShapeDtypeStruct((NC, NPAD, H), jnp.float32),
        ],
        mesh=mesh,
        compiler_params=pltpu.CompilerParams(use_tc_tiling_on_sc=False),
        scratch_types=[
            pltpu.VMEM((CHUNK,), jnp.int32),
            pltpu.VMEM((CHUNK,), jnp.int32),
            pltpu.VMEM((CHUNK,), jnp.int32),
            pltpu.VMEM((CHUNK,), jnp.int32),
            pltpu.VMEM((CHUNK,), jnp.int32),
            pltpu.VMEM((CHUNK,), jnp.int32),
            pltpu.VMEM((CHUNK,), jnp.int32),
            pltpu.VMEM((CHUNK,), jnp.int32),
            pltpu.VMEM((CHUNK, H), jnp.float32),
            pltpu.VMEM((CHUNK, H), jnp.float32),
            pltpu.VMEM((CHUNK, H), jnp.float32),
            pltpu.VMEM((CHUNK, H), jnp.float32),
            pltpu.VMEM((CHUNK, H), jnp.float32),
            pltpu.VMEM((CHUNK, H), jnp.float32),
            pltpu.VMEM((H,), jnp.float32),
            pltpu.VMEM_SHARED((NPAD, H), jnp.float32),
            pltpu.SemaphoreType.DMA,
            pltpu.SemaphoreType.DMA,
            pltpu.SemaphoreType.DMA,
            pltpu.SemaphoreType.DMA,
            pltpu.SemaphoreType.DMA,
            pltpu.SemaphoreType.DMA,
            pltpu.SemaphoreType.DMA,
            pltpu.SemaphoreType.DMA,
        ],
    )(_k2_body)


def _k3_body(src2, dst2, ex_hbm, den_hbm, h1a_hbm, h1b_hbm, z128_hbm,
             agg_hbm,
             sidx0, sidx1, sidx2, sidx3, didx0, didx1, didx2, didx3,
             rows0, rows1, exv0, exv1, slab,
             isem0, isem1, isem2, isem3, gsem0, gsem1, ssem0, ssem1):
    c = lax.axis_index("c")
    s = lax.axis_index("s")
    base = s * K3_RPS
    pltpu.sync_copy(z128_hbm, slab.at[pl.ds(s * NPS, NPS)])
    plsc.subcore_barrier()

    sidx_b = (sidx0, sidx1, sidx2, sidx3)
    didx_b = (didx0, didx1, didx2, didx3)
    isem_b = (isem0, isem1, isem2, isem3)
    rows_b = (rows0, rows1)
    exv_b = (exv0, exv1)
    gsem_b = (gsem0, gsem1)
    ssem_b = (ssem0, ssem1)

    def idx_load(j, q):
        pltpu.async_copy(src2.at[base + j, 0], sidx_b[q], isem_b[q])
        pltpu.async_copy(dst2.at[base + j, 0], didx_b[q], isem_b[q])

    def wait_idx(q):
        pltpu.make_async_copy(src2.at[base, 0], sidx_b[q], isem_b[q]).wait()
        pltpu.make_async_copy(dst2.at[base, 0], didx_b[q], isem_b[q]).wait()

    def gathers(j, d, q):
        dst = rows_b[d].at[pl.ds(0, CHUNK)]

        @pl.when(c == 0)
        def _():
            pltpu.async_copy(h1a_hbm.at[sidx_b[q]], dst, gsem_b[d])

        @pl.when(c == 1)
        def _():
            pltpu.async_copy(h1b_hbm.at[sidx_b[q]], dst, gsem_b[d])

        pltpu.async_copy(ex_hbm.at[base + j], exv_b[d].at[pl.ds(0, CHUNK)],
                         gsem_b[d])

    def wait_g(d, q):
        pltpu.make_async_copy(h1a_hbm.at[sidx_b[q]],
                              rows_b[d].at[pl.ds(0, CHUNK)],
                              gsem_b[d]).wait()
        pltpu.make_async_copy(ex_hbm.at[base], exv_b[d].at[pl.ds(0, CHUNK)],
                              gsem_b[d]).wait()

    def scatter(d, q):
        return pltpu.async_copy(rows_b[d].at[pl.ds(0, CHUNK)],
                                slab.at[didx_b[q]], ssem_b[d], add=True)

    idx_load(0, 0)
    idx_load(1, 1)
    wait_idx(0)
    gathers(0, 0, 0)

    def quad(jj, _):
        pend = []
        for t in range(4):
            j = 4 * jj + t
            d = t % 2
            u = 1 - d
            rows, exv = rows_b[d], exv_b[d]
            wait_g(d, t)

            @plsc.parallel_loop(0, CHUNK, unroll=5)
            def _(e):
                w = exv[e]
                for qf in range(DH // 16):
                    sl = pl.ds(16 * qf, 16)
                    rows[e, sl] = rows[e, sl] * w

            if t == 0:
                @pl.when(jj > 0)
                def _():
                    wait_idx(1)
                    gathers(j + 1, u, 1)
                    idx_load(j + 2, 2)

                @pl.when(jj == 0)
                def _():
                    wait_idx(1)
                    gathers(j + 1, u, 1)
                    idx_load(j + 2, 2)
            elif t == 3:
                pend[2].wait()

                @pl.when(jj < K3_RPS // 4 - 1)
                def _():
                    wait_idx(0)
                    gathers(j + 1, u, 0)
                    idx_load(j + 2, 1)
            else:
                pend[t - 1].wait()
                wait_idx(t + 1)
                gathers(j + 1, u, t + 1)
                if t == 2:
                    @pl.when(jj < K3_RPS // 4 - 1)
                    def _():
                        idx_load(j + 2, 0)
                else:
                    idx_load(j + 2, 3)

            pend.append(scatter(d, t))
        pend[3].wait()
        return 0

    lax.fori_loop(0, K3_RPS // 4, quad, 0)
    plsc.subcore_barrier()

    outbuf, d0, d1 = rows0, exv0, exv1

    def outchunk(k, _):
        nbase = s * NPS + k * OUTCH
        pltpu.sync_copy(slab.at[pl.ds(nbase, OUTCH)], outbuf)
        pltpu.sync_copy(den_hbm.at[0, pl.ds(nbase, OUTCH)], d0)
        pltpu.sync_copy(den_hbm.at[1, pl.ds(nbase, OUTCH)], d1)

        @plsc.parallel_loop(0, OUTCH, unroll=4)
        def _(e):
            dsum = d0[e] + d1[e]
            rd = jnp.where(dsum > 0, 1.0 / dsum, 0.0)
            for q in range(DH // 16):
                sl = pl.ds(16 * q, 16)
                outbuf[e, sl] = outbuf[e, sl] * rd

        pltpu.sync_copy(outbuf, agg_hbm.at[c, pl.ds(nbase, OUTCH)])
        return 0

    lax.fori_loop(0, NPS // OUTCH, outchunk, 0)


@functools.lru_cache(maxsize=None)
def _k3():
    mesh = plsc.VectorSubcoreMesh(core_axis_name="c", subcore_axis_name="s")
    return functools.partial(
        pl.kernel,
        out_type=jax.ShapeDtypeStruct((NC, NPAD, DH), jnp.float32),
        mesh=mesh,
        compiler_params=pltpu.CompilerParams(use_tc_tiling_on_sc=False),
        scratch_types=[
            pltpu.VMEM((CHUNK,), jnp.int32),
            pltpu.VMEM((CHUNK,), jnp.int32),
            pltpu.VMEM((CHUNK,), jnp.int32),
            pltpu.VMEM((CHUNK,), jnp.int32),
            pltpu.VMEM((CHUNK,), jnp.int32),
            pltpu.VMEM((CHUNK,), jnp.int32),
            pltpu.VMEM((CHUNK,), jnp.int32),
            pltpu.VMEM((CHUNK,), jnp.int32),
            pltpu.VMEM((OUTCH, DH), jnp.float32),
            pltpu.VMEM((OUTCH, DH), jnp.float32),
            pltpu.VMEM((OUTCH, H), jnp.float32),
            pltpu.VMEM((OUTCH, H), jnp.float32),
            pltpu.VMEM_SHARED((NPAD, DH), jnp.float32),
            pltpu.SemaphoreType.DMA,
            pltpu.SemaphoreType.DMA,
            pltpu.SemaphoreType.DMA,
            pltpu.SemaphoreType.DMA,
            pltpu.SemaphoreType.DMA,
            pltpu.SemaphoreType.DMA,
            pltpu.SemaphoreType.DMA,
            pltpu.SemaphoreType.DMA,
        ],
    )(_k3_body)


def _k4_body(aggA_ref, aggB_ref, w1aT_ref, w1bT_ref, b1_ref, w2T_ref, b2_ref,
             out_ref):
    z = (jnp.dot(aggA_ref[...].astype(jnp.bfloat16), w1aT_ref[...],
                 preferred_element_type=jnp.float32)
         + jnp.dot(aggB_ref[...].astype(jnp.bfloat16), w1bT_ref[...],
                   preferred_element_type=jnp.float32)
         + b1_ref[...])
    z = 0.5 * z * (1.0 + lax.erf(z * 0.7071067811865476))
    out_ref[...] = jnp.dot(z.astype(jnp.bfloat16), w2T_ref[...],
                           preferred_element_type=jnp.float32) + b2_ref[...]


def _k4(aggA, aggB, w1aT, w1bT, b12, w2T, b22, hidden):
    nblk = N // ROW_BLK
    return pl.pallas_call(
        _k4_body,
        grid=(nblk,),
        in_specs=[
            pl.BlockSpec((ROW_BLK, DH), lambda i: (i, 0)),
            pl.BlockSpec((ROW_BLK, DH), lambda i: (i, 0)),
            pl.BlockSpec((DH, hidden), lambda i: (0, 0)),
            pl.BlockSpec((DH, hidden), lambda i: (0, 0)),
            pl.BlockSpec((1, hidden), lambda i: (0, 0)),
            pl.BlockSpec((hidden, D), lambda i: (0, 0)),
            pl.BlockSpec((1, D), lambda i: (0, 0)),
        ],
        out_specs=pl.BlockSpec((ROW_BLK, D), lambda i: (i, 0)),
        out_shape=jax.ShapeDtypeStruct((N, D), jnp.float32),
    )(aggA, aggB, w1aT, w1bT, b12, w2T, b22)


def kernel(h, edge_index, W_in, b_in, W_u, b_u, W_v, W1, b1, W2, b2):
    hidden = W1.shape[0]
    winT = W_in.T
    bin2 = b_in.reshape(1, D)
    wuv = jnp.concatenate([jnp.tile(W_u.T, (1, 2)), jnp.tile(W_v.T, (1, 2))],
                          axis=1)
    buv = jnp.concatenate([jnp.tile(b_u, 2), jnp.zeros((H,), jnp.float32)]
                          ).reshape(1, 2 * H)
    src2 = edge_index[0].reshape(NROWS, 1, CHUNK)
    dst2 = edge_index[1].reshape(NROWS, 1, CHUNK)
    z16 = jnp.zeros((NPS, H), jnp.float32)
    z128 = jnp.zeros((NPS, DH), jnp.float32)
    w1aT = W1[:, :DH].T.astype(jnp.bfloat16)
    w1bT = W1[:, DH:].T.astype(jnp.bfloat16)
    b12 = b1.reshape(1, hidden)
    w2T = W2.T.astype(jnp.bfloat16)
    b22 = b2.reshape(1, D)

    h1a, h1b, su_t, sv_t, m_t = _k1(h, winT, bin2, wuv, buv)
    ex, den = _k2()(src2, dst2, su_t, sv_t, m_t, z16)
    agg = _k3()(src2, dst2, ex, den, h1a, h1b, z128)
    out = _k4(agg[0], agg[1], w1aT, w1bT, b12, w2T, b22, hidden)
    return out


# speedup vs baseline: 1.0028x; 1.0028x over previous
import functools

import jax
import jax.numpy as jnp
from jax import lax
from jax.experimental import pallas as pl
from jax.experimental.pallas import tpu as pltpu
from jax.experimental.pallas import tpu_sc as plsc

N = 10000
E = 160000
D = 256
H = 16
DH = 128

NC = 2
NS = 16
NW = NC * NS
CHUNK = 125
NROWS = E // CHUNK
ROWS_PER_W = NROWS // NW
K3_RPS = NROWS // NS
NPAD = 10240
NPS = NPAD // NS
OUTCH = 128
ROW_BLK = 1000


def _k1_body(h_ref, winT_ref, bin_ref, wuv_ref, buv_ref,
             h1a_ref, h1b_ref, su_ref, sv_ref, m_ref, acc_ref):
    i = pl.program_id(0)
    h1 = lax.dot_general(h_ref[...], winT_ref[...],
                         (((1,), (1,)), ((), ())),
                         preferred_element_type=jnp.float32) + bin_ref[...]
    h1a_ref[...] = h1[:, :DH]
    h1b_ref[...] = h1[:, DH:]
    suv = jnp.dot(h1, wuv_ref[...],
                  preferred_element_type=jnp.float32) + buv_ref[...]
    su_ref[...] = suv[:, :H]
    sv_ref[...] = suv[:, H:]
    blkmax = jnp.max(suv, axis=0, keepdims=True)

    @pl.when(i == 0)
    def _():
        acc_ref[...] = blkmax

    @pl.when(i > 0)
    def _():
        acc_ref[...] = jnp.maximum(acc_ref[...], blkmax)

    m = acc_ref[...]
    mm = m[:, :H] + m[:, H:]
    m_ref[...] = jnp.where(mm >= 0, mm, 0.2 * mm)


def _k1(h, winT, bin2, wuv, buv):
    nblk = N // ROW_BLK
    return pl.pallas_call(
        _k1_body,
        grid=(nblk,),
        in_specs=[
            pl.BlockSpec((ROW_BLK, D), lambda i: (i, 0)),
            pl.BlockSpec((D, D), lambda i: (0, 0)),
            pl.BlockSpec((1, D), lambda i: (0, 0)),
            pl.BlockSpec((D, 2 * H), lambda i: (0, 0)),
            pl.BlockSpec((1, 2 * H), lambda i: (0, 0)),
        ],
        out_specs=[
            pl.BlockSpec((ROW_BLK, DH), lambda i: (i, 0)),
            pl.BlockSpec((ROW_BLK, DH), lambda i: (i, 0)),
            pl.BlockSpec((ROW_BLK, H), lambda i: (i, 0)),
            pl.BlockSpec((ROW_BLK, H), lambda i: (i, 0)),
            pl.BlockSpec((1, H), lambda i: (0, 0)),
        ],
        out_shape=[
            jax.ShapeDtypeStruct((N, DH), jnp.float32),
            jax.ShapeDtypeStruct((N, DH), jnp.float32),
            jax.ShapeDtypeStruct((N, H), jnp.float32),
            jax.ShapeDtypeStruct((N, H), jnp.float32),
            jax.ShapeDtypeStruct((1, H), jnp.float32),
        ],
        scratch_shapes=[pltpu.VMEM((1, 2 * H), jnp.float32)],
    )(h, winT, bin2, wuv, buv)


def _k2_body(src2, dst2, su_hbm, sv_hbm, m_hbm, z16_hbm,
             ex_hbm, den_hbm,
             sidx0, sidx1, sidx2, sidx3, didx0, didx1, didx2, didx3,
             surow0, svrow0, p0, surow1, svrow1, p1,
             mvec, den_sh,
             isem0, isem1, isem2, isem3, gsem0, gsem1, wsem0, wsem1):
    c = lax.axis_index("c")
    s = lax.axis_index("s")
    wid = c * NS + s
    base = wid * ROWS_PER_W
    pltpu.sync_copy(z16_hbm, den_sh.at[pl.ds(s * NPS, NPS)])
    pltpu.sync_copy(m_hbm.at[0], mvec)
    plsc.subcore_barrier()
    mval = mvec[...]

    sidx_b = (sidx0, sidx1, sidx2, sidx3)
    didx_b = (didx0, didx1, didx2, didx3)
    isem_b = (isem0, isem1, isem2, isem3)
    surow_b = (surow0, surow1)
    svrow_b = (svrow0, svrow1)
    p_b = (p0, p1)
    gsem_b = (gsem0, gsem1)
    wsem_b = (wsem0, wsem1)

    def idx_load(j, q):
        pltpu.async_copy(src2.at[base + j, 0], sidx_b[q], isem_b[q])
        pltpu.async_copy(dst2.at[base + j, 0], didx_b[q], isem_b[q])

    def wait_idx(q):
        pltpu.make_async_copy(src2.at[base, 0], sidx_b[q], isem_b[q]).wait()
        pltpu.make_async_copy(dst2.at[base, 0], didx_b[q], isem_b[q]).wait()

    def gathers(d, q):
        pltpu.async_copy(su_hbm.at[sidx_b[q]], surow_b[d], gsem_b[d])
        pltpu.async_copy(sv_hbm.at[didx_b[q]], svrow_b[d], gsem_b[d])

    def wait_g(d, q):
        pltpu.make_async_copy(su_hbm.at[sidx_b[q]], surow_b[d],
                              gsem_b[d]).wait()
        pltpu.make_async_copy(sv_hbm.at[didx_b[q]], svrow_b[d],
                              gsem_b[d]).wait()

    def stores(j, d, q):
        pltpu.async_copy(p_b[d], ex_hbm.at[base + j], wsem_b[d])
        pltpu.sync_copy(p_b[d], den_sh.at[didx_b[q]], add=True)

    def wait_w(d):
        pltpu.make_async_copy(p_b[d], ex_hbm.at[base], wsem_b[d]).wait()

    idx_load(0, 0)
    idx_load(1, 1)
    wait_idx(0)
    gathers(0, 0)

    def quad(jj, _):
        for t in range(4):
            j = 4 * jj + t
            d = t % 2
            u = 1 - d
            surow, svrow, p = surow_b[d], svrow_b[d], p_b[d]
            wait_g(d, t)
            if t < 2:
                @pl.when(jj > 0)
                def _():
                    wait_w(d)
            else:
                wait_w(d)

            @plsc.parallel_loop(0, CHUNK, unroll=5)
            def _(e):
                v = surow[e] + svrow[e]
                v = jnp.where(v >= 0, v, 0.2 * v) - mval
                p[e] = jnp.exp(v)

            if t == 0:
                wait_idx(1)
                gathers(u, 1)
                idx_load(j + 2, 2)
            elif t == 3:
                @pl.when(jj < ROWS_PER_W // 4 - 1)
                def _():
                    wait_idx(0)
                    gathers(u, 0)
                    idx_load(j + 2, 1)
            else:
                wait_idx(t + 1)
                gathers(u, t + 1)
                if t == 2:
                    @pl.when(jj < ROWS_PER_W // 4 - 1)
                    def _():
                        idx_load(j + 2, 0)
                else:
                    idx_load(j + 2, 3)

            stores(j, d, t)
        return 0

    lax.fori_loop(0, ROWS_PER_W // 4, quad, 0)
    wait_w(0)
    wait_w(1)
    plsc.subcore_barrier()
    pltpu.sync_copy(den_sh.at[pl.ds(s * NPS, NPS)],
                    den_hbm.at[c, pl.ds(s * NPS, NPS)])


@functools.lru_cache(maxsize=None)
def _k2():
    mesh = plsc.VectorSubcoreMesh(core_axis_name="c", subcore_axis_name="s")
    return functools.partial(
        pl.kernel,
        out_type=[
            jax.ShapeDtypeStruct((NROWS, CHUNK, H), jnp.float32),
            jax.ShapeDtypeStruct((NC, NPAD, H), jnp.float32),
        ],
        mesh=mesh,
        compiler_params=pltpu.CompilerParams(use_tc_tiling_on_sc=False),
        scratch_types=[
            pltpu.VMEM((CHUNK,), jnp.int32),
            pltpu.VMEM((CHUNK,), jnp.int32),
            pltpu.VMEM((CHUNK,), jnp.int32),
            pltpu.VMEM((CHUNK,), jnp.int32),
            pltpu.VMEM((CHUNK,), jnp.int32),
            pltpu.VMEM((CHUNK,), jnp.int32),
            pltpu.VMEM((CHUNK,), jnp.int32),
            pltpu.VMEM((CHUNK,), jnp.int32),
            pltpu.VMEM((CHUNK, H), jnp.float32),
            pltpu.VMEM((CHUNK, H), jnp.float32),
            pltpu.VMEM((CHUNK, H), jnp.float32),
            pltpu.VMEM((CHUNK, H), jnp.float32),
            pltpu.VMEM((CHUNK, H), jnp.float32),
            pltpu.VMEM((CHUNK, H), jnp.float32),
            pltpu.VMEM((H,), jnp.float32),
            pltpu.VMEM_SHARED((NPAD, H), jnp.float32),
            pltpu.SemaphoreType.DMA,
            pltpu.SemaphoreType.DMA,
            pltpu.SemaphoreType.DMA,
            pltpu.SemaphoreType.DMA,
            pltpu.SemaphoreType.DMA,
            pltpu.SemaphoreType.DMA,
            pltpu.SemaphoreType.DMA,
            pltpu.SemaphoreType.DMA,
        ],
    )(_k2_body)


def _k3_body(src2, dst2, ex_hbm, den_hbm, h1a_hbm, h1b_hbm, z128_hbm,
             agg_hbm,
             sidx0, sidx1, sidx2, sidx3, didx0, didx1, didx2, didx3,
             rows0, rows1, exv0, exv1, slab,
             isem0, isem1, isem2, isem3, gsem0, gsem1, ssem0, ssem1):
    c = lax.axis_index("c")
    s = lax.axis_index("s")
    base = s * K3_RPS
    pltpu.sync_copy(z128_hbm, slab.at[pl.ds(s * NPS, NPS)])
    plsc.subcore_barrier()

    sidx_b = (sidx0, sidx1, sidx2, sidx3)
    didx_b = (didx0, didx1, didx2, didx3)
    isem_b = (isem0, isem1, isem2, isem3)
    rows_b = (rows0, rows1)
    exv_b = (exv0, exv1)
    gsem_b = (gsem0, gsem1)
    ssem_b = (ssem0, ssem1)

    def idx_load(j, q):
        pltpu.async_copy(src2.at[base + j, 0], sidx_b[q], isem_b[q])
        pltpu.async_copy(dst2.at[base + j, 0], didx_b[q], isem_b[q])

    def wait_idx(q):
        pltpu.make_async_copy(src2.at[base, 0], sidx_b[q], isem_b[q]).wait()
        pltpu.make_async_copy(dst2.at[base, 0], didx_b[q], isem_b[q]).wait()

    def gathers(j, d, q):
        dst = rows_b[d].at[pl.ds(0, CHUNK)]

        @pl.when(c == 0)
        def _():
            pltpu.async_copy(h1a_hbm.at[sidx_b[q]], dst, gsem_b[d])

        @pl.when(c == 1)
        def _():
            pltpu.async_copy(h1b_hbm.at[sidx_b[q]], dst, gsem_b[d])

        pltpu.async_copy(ex_hbm.at[base + j], exv_b[d].at[pl.ds(0, CHUNK)],
                         gsem_b[d])

    def wait_g(d, q):
        pltpu.make_async_copy(h1a_hbm.at[sidx_b[q]],
                              rows_b[d].at[pl.ds(0, CHUNK)],
                              gsem_b[d]).wait()
        pltpu.make_async_copy(ex_hbm.at[base], exv_b[d].at[pl.ds(0, CHUNK)],
                              gsem_b[d]).wait()

    def scatter(d, q):
        return pltpu.async_copy(rows_b[d].at[pl.ds(0, CHUNK)],
                                slab.at[didx_b[q]], ssem_b[d], add=True)

    idx_load(0, 0)
    idx_load(1, 1)
    wait_idx(0)
    gathers(0, 0, 0)

    def quad(jj, _):
        pend = []
        for t in range(4):
            j = 4 * jj + t
            d = t % 2
            u = 1 - d
            rows, exv = rows_b[d], exv_b[d]
            wait_g(d, t)

            @plsc.parallel_loop(0, CHUNK, unroll=5)
            def _(e):
                w = exv[e]
                for qf in range(DH // 16):
                    sl = pl.ds(16 * qf, 16)
                    rows[e, sl] = rows[e, sl] * w

            if t == 0:
                @pl.when(jj > 0)
                def _():
                    wait_idx(1)
                    gathers(j + 1, u, 1)
                    idx_load(j + 2, 2)

                @pl.when(jj == 0)
                def _():
                    wait_idx(1)
                    gathers(j + 1, u, 1)
                    idx_load(j + 2, 2)
            elif t == 3:
                pend[2].wait()

                @pl.when(jj < K3_RPS // 4 - 1)
                def _():
                    wait_idx(0)
                    gathers(j + 1, u, 0)
                    idx_load(j + 2, 1)
            else:
                pend[t - 1].wait()
                wait_idx(t + 1)
                gathers(j + 1, u, t + 1)
                if t == 2:
                    @pl.when(jj < K3_RPS // 4 - 1)
                    def _():
                        idx_load(j + 2, 0)
                else:
                    idx_load(j + 2, 3)

            pend.append(scatter(d, t))
        pend[3].wait()
        return 0

    lax.fori_loop(0, K3_RPS // 4, quad, 0)
    plsc.subcore_barrier()

    outbuf, d0, d1 = rows0, exv0, exv1

    def outchunk(k, _):
        nbase = s * NPS + k * OUTCH
        pltpu.sync_copy(slab.at[pl.ds(nbase, OUTCH)], outbuf)
        pltpu.sync_copy(den_hbm.at[0, pl.ds(nbase, OUTCH)], d0)
        pltpu.sync_copy(den_hbm.at[1, pl.ds(nbase, OUTCH)], d1)

        @plsc.parallel_loop(0, OUTCH, unroll=4)
        def _(e):
            dsum = d0[e] + d1[e]
            rd = jnp.where(dsum > 0, 1.0 / dsum, 0.0)
            for q in range(DH // 16):
                sl = pl.ds(16 * q, 16)
                outbuf[e, sl] = outbuf[e, sl] * rd

        pltpu.sync_copy(outbuf, agg_hbm.at[c, pl.ds(nbase, OUTCH)])
        return 0

    lax.fori_loop(0, NPS // OUTCH, outchunk, 0)


@functools.lru_cache(maxsize=None)
def _k3():
    mesh = plsc.VectorSubcoreMesh(core_axis_name="c", subcore_axis_name="s")
    return functools.partial(
        pl.kernel,
        out_type=jax.ShapeDtypeStruct((NC, NPAD, DH), jnp.float32),
        mesh=mesh,
        compiler_params=pltpu.CompilerParams(use_tc_tiling_on_sc=False),
        scratch_types=[
            pltpu.VMEM((CHUNK,), jnp.int32),
            pltpu.VMEM((CHUNK,), jnp.int32),
            pltpu.VMEM((CHUNK,), jnp.int32),
            pltpu.VMEM((CHUNK,), jnp.int32),
            pltpu.VMEM((CHUNK,), jnp.int32),
            pltpu.VMEM((CHUNK,), jnp.int32),
            pltpu.VMEM((CHUNK,), jnp.int32),
            pltpu.VMEM((CHUNK,), jnp.int32),
            pltpu.VMEM((OUTCH, DH), jnp.float32),
            pltpu.VMEM((OUTCH, DH), jnp.float32),
            pltpu.VMEM((OUTCH, H), jnp.float32),
            pltpu.VMEM((OUTCH, H), jnp.float32),
            pltpu.VMEM_SHARED((NPAD, DH), jnp.float32),
            pltpu.SemaphoreType.DMA,
            pltpu.SemaphoreType.DMA,
            pltpu.SemaphoreType.DMA,
            pltpu.SemaphoreType.DMA,
            pltpu.SemaphoreType.DMA,
            pltpu.SemaphoreType.DMA,
            pltpu.SemaphoreType.DMA,
            pltpu.SemaphoreType.DMA,
        ],
    )(_k3_body)


def _k4_body(aggA_ref, aggB_ref, w1aT_ref, w1bT_ref, b1_ref, w2T_ref, b2_ref,
             out_ref):
    dn = (((1,), (1,)), ((), ()))
    z = (lax.dot_general(aggA_ref[...].astype(jnp.bfloat16), w1aT_ref[...],
                         dn, preferred_element_type=jnp.float32)
         + lax.dot_general(aggB_ref[...].astype(jnp.bfloat16), w1bT_ref[...],
                           dn, preferred_element_type=jnp.float32)
         + b1_ref[...])
    z = 0.5 * z * (1.0 + lax.erf(z * 0.7071067811865476))
    out_ref[...] = lax.dot_general(z.astype(jnp.bfloat16), w2T_ref[...],
                                   dn,
                                   preferred_element_type=jnp.float32
                                   ) + b2_ref[...]


def _k4(aggA, aggB, w1aT, w1bT, b12, w2T, b22, hidden):
    nblk = N // ROW_BLK
    return pl.pallas_call(
        _k4_body,
        grid=(nblk,),
        in_specs=[
            pl.BlockSpec((ROW_BLK, DH), lambda i: (i, 0)),
            pl.BlockSpec((ROW_BLK, DH), lambda i: (i, 0)),
            pl.BlockSpec((hidden, DH), lambda i: (0, 0)),
            pl.BlockSpec((hidden, DH), lambda i: (0, 0)),
            pl.BlockSpec((1, hidden), lambda i: (0, 0)),
            pl.BlockSpec((D, hidden), lambda i: (0, 0)),
            pl.BlockSpec((1, D), lambda i: (0, 0)),
        ],
        out_specs=pl.BlockSpec((ROW_BLK, D), lambda i: (i, 0)),
        out_shape=jax.ShapeDtypeStruct((N, D), jnp.float32),
    )(aggA, aggB, w1aT, w1bT, b12, w2T, b22)


def kernel(h, edge_index, W_in, b_in, W_u, b_u, W_v, W1, b1, W2, b2):
    hidden = W1.shape[0]
    winT = W_in
    bin2 = b_in.reshape(1, D)
    wuv = jnp.concatenate([jnp.tile(W_u.T, (1, 2)), jnp.tile(W_v.T, (1, 2))],
                          axis=1)
    buv = jnp.concatenate([jnp.tile(b_u, 2), jnp.zeros((H,), jnp.float32)]
                          ).reshape(1, 2 * H)
    src2 = edge_index[0].reshape(NROWS, 1, CHUNK)
    dst2 = edge_index[1].reshape(NROWS, 1, CHUNK)
    z16 = jnp.zeros((NPS, H), jnp.float32)
    z128 = jnp.zeros((NPS, DH), jnp.float32)
    w1aT = W1[:, :DH].astype(jnp.bfloat16)
    w1bT = W1[:, DH:].astype(jnp.bfloat16)
    b12 = b1.reshape(1, hidden)
    w2T = W2.astype(jnp.bfloat16)
    b22 = b2.reshape(1, D)

    h1a, h1b, su_t, sv_t, m_t = _k1(h, winT, bin2, wuv, buv)
    ex, den = _k2()(src2, dst2, su_t, sv_t, m_t, z16)
    agg = _k3()(src2, dst2, ex, den, h1a, h1b, z128)
    out = _k4(agg[0], agg[1], w1aT, w1bT, b12, w2T, b22, hidden)
    return out
